# baseline (speedup 1.0000x reference)
import functools

import jax
import jax.numpy as jnp
import numpy as np
from jax import lax
from jax.experimental import pallas as pl
from jax.experimental.pallas import tpu as pltpu
from jax.experimental.pallas import tpu_sc as plsc

N = 10000
E = 320000
HID = 128
XW = 8
NUM_G = 16
LANES = 16

NC = 2
NS = 16
NW = NC * NS
EPW = E // NW
SUB = 80
NSUB = 5
CHUNK = SUB * NSUB
NCHUNK = EPW // CHUNK
NCHUNKS_ALL = E // CHUNK
S_CHUNK = 80
S_NCHUNK = EPW // S_CHUNK
S_NCHUNKS_ALL = E // S_CHUNK
NACC = 10240
NGRP = NACC // LANES
NACC2 = NACC + NGRP
ROWS_PT = NACC2 // NS


def _pre_body(h_ref, wa_ref, wb_ref, a_ref, b_ref):
    h = h_ref[...]
    a_ref[...] = jnp.dot(h, wa_ref[...], preferred_element_type=jnp.float32)
    b_ref[...] = jnp.dot(h, wb_ref[...], preferred_element_type=jnp.float32)


def _tc_pre(h, We1a, We1b):
    return pl.pallas_call(
        _pre_body,
        out_shape=(jax.ShapeDtypeStruct((N, HID), jnp.float32),
                   jax.ShapeDtypeStruct((N, HID), jnp.float32)),
    )(h, We1a, We1b)


_EB = 2560
_EGRID = E // _EB
_G_STEP = float(np.float32(10.0) / np.float32(NUM_G - 1))
_G_COEFF = float(-0.5 / np.linspace(0.0, 10.0, NUM_G)[1] ** 2)


def _edge_body(ad_ref, bs_ref, rel_ref, ea_ref, wd_ref, wea_ref, be1_ref,
               we2_ref, be2_ref, winf_ref, binf_ref, wx1_ref, bx1_ref,
               wx2_ref, msg_ref, xmsg_ref):
    t1pre = ad_ref[...] + bs_ref[...]
    rel = rel_ref[...]
    d_sq = jnp.sum(rel * rel, axis=1, keepdims=True)
    dist = jnp.sqrt(d_sq + 1e-8)
    offs = (lax.broadcasted_iota(jnp.int32, (1, NUM_G), 1)
            .astype(jnp.float32) * _G_STEP)
    dfeat = jnp.exp(_G_COEFF * (dist - offs) ** 2)
    t1 = (t1pre
          + jnp.dot(dfeat, wd_ref[...], preferred_element_type=jnp.float32)
          + be1_ref[...])
    ea = ea_ref[...]
    wea = wea_ref[...]
    for k in range(4):
        t1 = t1 + ea[:, k:k + 1] * wea[k:k + 1, :]
    u = t1 * jax.nn.sigmoid(t1)
    m1 = jnp.dot(u, we2_ref[...], preferred_element_type=jnp.float32) + be2_ref[...]
    mij = m1 * jax.nn.sigmoid(m1)
    eij = jax.nn.sigmoid(
        jnp.sum(mij * winf_ref[...], axis=1, keepdims=True) + binf_ref[...])
    v1 = jnp.dot(mij, wx1_ref[...], preferred_element_type=jnp.float32) + bx1_ref[...]
    v = v1 * jax.nn.sigmoid(v1)
    xg = jnp.tanh(jnp.sum(v * wx2_ref[...], axis=1, keepdims=True))
    xmsg = rel * (xg / (dist + 1.0))
    msg_ref[...] = mij * eij
    xmsg_ref[...] = xmsg


def _tc_edge(ad, bs, rel, edge_attr, We1d, We1e, be1, We2, be2, winf_row,
             binf, Wx1, bx1, wx2_row):
    full = lambda shape: pl.BlockSpec(shape, lambda i: (0, 0))
    return pl.pallas_call(
        _edge_body,
        grid=(_EGRID,),
        in_specs=[
            pl.BlockSpec((_EB, HID), lambda i: (i, 0)),
            pl.BlockSpec((_EB, HID), lambda i: (i, 0)),
            pl.BlockSpec((_EB, XW), lambda i: (i, 0)),
            pl.BlockSpec((_EB, 4), lambda i: (i, 0)),
            full((NUM_G, HID)),
            full((4, HID)),
            full((1, HID)),
            full((HID, HID)),
            full((1, HID)),
            full((1, HID)),
            full((1, 1)),
            full((HID, HID)),
            full((1, HID)),
            full((1, HID)),
        ],
        out_specs=(pl.BlockSpec((_EB, HID), lambda i: (i, 0)),
                   pl.BlockSpec((_EB, XW), lambda i: (i, 0))),
        out_shape=(jax.ShapeDtypeStruct((E, HID), jnp.float32),
                   jax.ShapeDtypeStruct((E, XW), jnp.float32)),
    )(ad, bs, rel, edge_attr, We1d, We1e, be1, We2, be2, winf_row, binf,
      Wx1, bx1, wx2_row)


def _node_body(h_ref, xp_ref, pm_ref, pd_ref, mask_ref, wn1a_ref, wn1b_ref,
               bn1_ref, wn2_ref, bn2_ref, hout_ref, xout_ref):
    h = h_ref[...]
    mi = pm_ref[0] + pm_ref[1]
    dx = pd_ref[0] + pd_ref[1]
    t1 = (jnp.dot(mi, wn1a_ref[...], preferred_element_type=jnp.float32)
          + jnp.dot(h, wn1b_ref[...], preferred_element_type=jnp.float32)
          + bn1_ref[...])
    t = t1 * jax.nn.sigmoid(t1)
    hout_ref[...] = h + jnp.dot(t, wn2_ref[...],
                                preferred_element_type=jnp.float32) + bn2_ref[...]
    xout_ref[...] = xp_ref[...] + dx * mask_ref[...]


def _tc_node(h, xpad, parts_msg, parts_dx, mask_f, Wn1a, Wn1b, bn1, Wn2, bn2):
    return pl.pallas_call(
        _node_body,
        out_shape=(jax.ShapeDtypeStruct((N, HID), jnp.float32),
                   jax.ShapeDtypeStruct((N, XW), jnp.float32)),
    )(h, xpad, parts_msg, parts_dx, mask_f, Wn1a, Wn1b, bn1, Wn2, bn2)


@functools.cache
def _sc_gather_kernel():
    mesh = plsc.VectorSubcoreMesh(core_axis_name="c", subcore_axis_name="s")
    return functools.partial(
        pl.kernel,
        mesh=mesh,
        out_type=(jax.ShapeDtypeStruct((E, HID), jnp.float32),
                  jax.ShapeDtypeStruct((E, HID), jnp.float32)),
        scratch_types=[
            pltpu.VMEM((NSUB, SUB), jnp.int32),
            pltpu.VMEM((NSUB, SUB), jnp.int32),
            pltpu.VMEM((CHUNK, HID), jnp.float32),
            pltpu.VMEM((CHUNK, HID), jnp.float32),
            pltpu.SemaphoreType.DMA,
        ],
    )(_sc_gather_body)


def _sc_gather(a, b, dst3d, src3d):
    return _sc_gather_kernel()(a, b, dst3d, src3d)


def _sc_gather_body(a_hbm, b_hbm, dst3d_hbm, src3d_hbm, ad_out, bs_out,
                    idxd, idxs, adb, bsb, sem):
    c = lax.axis_index("c")
    s = lax.axis_index("s")
    wid = s * NC + c
    g0 = wid * NCHUNK
    e0w = wid * EPW

    def chunk(k, carry):
        pltpu.sync_copy(dst3d_hbm.at[g0 + k], idxd)
        pltpu.sync_copy(src3d_hbm.at[g0 + k], idxs)
        copies = []
        for j in range(NSUB):
            copies.append(pltpu.async_copy(
                a_hbm.at[idxd.at[j]], adb.at[pl.ds(j * SUB, SUB)], sem))
            copies.append(pltpu.async_copy(
                b_hbm.at[idxs.at[j]], bsb.at[pl.ds(j * SUB, SUB)], sem))
        for cp in copies:
            cp.wait()
        e0 = e0w + k * CHUNK
        pltpu.sync_copy(adb, ad_out.at[pl.ds(e0, CHUNK)])
        pltpu.sync_copy(bsb, bs_out.at[pl.ds(e0, CHUNK)])
        return carry

    lax.fori_loop(0, NCHUNK, chunk, 0)


@functools.cache
def _sc_rel_kernel():
    mesh = plsc.VectorSubcoreMesh(core_axis_name="c", subcore_axis_name="s")
    return functools.partial(
        pl.kernel,
        mesh=mesh,
        compiler_params=pltpu.CompilerParams(needs_layout_passes=False),
        out_type=jax.ShapeDtypeStruct((E * XW,), jnp.float32),
        scratch_types=[
            pltpu.VMEM((N,), jnp.float32),
            pltpu.VMEM((N,), jnp.float32),
            pltpu.VMEM((N,), jnp.float32),
            pltpu.VMEM((NSUB, SUB), jnp.int32),
            pltpu.VMEM((NSUB, SUB), jnp.int32),
            pltpu.VMEM((CHUNK * XW,), jnp.float32),
            pltpu.SemaphoreType.DMA,
        ],
    )(_sc_rel_body)


def _sc_rel(x0, x1, x2, dst3d, src3d):
    return _sc_rel_kernel()(x0, x1, x2, dst3d, src3d)


def _sc_rel_body(x0_hbm, x1_hbm, x2_hbm, dst3d_hbm, src3d_hbm, rel_out,
                 x0b, x1b, x2b, idxd, idxs, relb, sem):
    c = lax.axis_index("c")
    s = lax.axis_index("s")
    wid = s * NC + c
    g0 = wid * NCHUNK
    e0w = wid * EPW
    pltpu.sync_copy(x0_hbm, x0b)
    pltpu.sync_copy(x1_hbm, x1b)
    pltpu.sync_copy(x2_hbm, x2b)

    def zero(v, carry):
        relb[pl.ds(v * LANES, LANES)] = jnp.zeros((LANES,), jnp.float32)
        return carry

    lax.fori_loop(0, CHUNK * XW // LANES, zero, 0)

    def chunk(k, carry):
        pltpu.sync_copy(dst3d_hbm.at[g0 + k], idxd)
        pltpu.sync_copy(src3d_hbm.at[g0 + k], idxs)
        for j in range(NSUB):
            for i in range(SUB // LANES):
                ivd = idxd[j, pl.ds(i * LANES, LANES)]
                ivs = idxs[j, pl.ds(i * LANES, LANES)]
                base = (j * SUB + i * LANES) * XW
                flat = lax.iota(jnp.int32, LANES) * XW + base
                for comp, xb in ((0, x0b), (1, x1b), (2, x2b)):
                    d = plsc.load_gather(xb, [ivd])
                    sv = plsc.load_gather(xb, [ivs])
                    plsc.store_scatter(relb, [flat + comp], d - sv)
        pltpu.sync_copy(relb, rel_out.at[pl.ds((e0w + k * CHUNK) * XW,
                                               CHUNK * XW)])
        return carry

    lax.fori_loop(0, NCHUNK, chunk, 0)


@functools.cache
def _sc_scatter_kernel():
    mesh = plsc.VectorSubcoreMesh(core_axis_name="c", subcore_axis_name="s")
    return functools.partial(
        pl.kernel,
        mesh=mesh,
        compiler_params=pltpu.CompilerParams(needs_layout_passes=False),
        out_type=jax.ShapeDtypeStruct((NC, NACC2, HID), jnp.float32),
        scratch_types=[
            pltpu.VMEM((1, S_CHUNK), jnp.int32),
            pltpu.VMEM((1, S_CHUNK), jnp.int32),
            pltpu.VMEM((S_CHUNK, HID), jnp.float32),
            pltpu.VMEM((S_CHUNK, XW), jnp.float32),
            pltpu.VMEM((S_CHUNK, HID), jnp.float32),
            pltpu.VMEM_SHARED((NACC2, HID), jnp.float32),
            pltpu.SemaphoreType.DMA,
        ],
    )(_sc_scatter_body)


def _sc_scatter(msg, xmsg, dst3d_s, zeros):
    return _sc_scatter_kernel()(msg, xmsg, dst3d_s, zeros)


def _sc_scatter_body(msg_hbm, xmsg_hbm, dst3d_hbm, zeros_hbm, out_hbm,
                     idxb, idxg, mbuf, mxbuf, xbuf, acc, sem):
    c = lax.axis_index("c")
    s = lax.axis_index("s")
    wid = c * NS + s
    g0 = wid * S_NCHUNK
    e0w = wid * EPW
    rows = pl.ds(s * ROWS_PT, ROWS_PT)

    pltpu.sync_copy(zeros_hbm.at[rows], acc.at[rows])

    def zero(v, carry):
        xbuf[v, pl.ds(0 * LANES, LANES)] = jnp.zeros((LANES,), jnp.float32)
        for q in range(1, HID // LANES):
            xbuf[v, pl.ds(q * LANES, LANES)] = jnp.zeros((LANES,), jnp.float32)
        return carry

    lax.fori_loop(0, S_CHUNK, zero, 0)
    plsc.subcore_barrier()

    zero16 = jnp.zeros((LANES,), jnp.float32)

    def chunk(k, carry):
        pltpu.sync_copy(dst3d_hbm.at[g0 + k], idxb)
        e0 = e0w + k * S_CHUNK
        pltpu.sync_copy(msg_hbm.at[pl.ds(e0, S_CHUNK)], mbuf)
        pltpu.sync_copy(xmsg_hbm.at[pl.ds(e0, S_CHUNK)], mxbuf)
        pltpu.sync_copy(mbuf, acc.at[idxb.at[0]], add=True)
        cols = []
        for i in range(S_CHUNK // LANES):
            dv = idxb[0, pl.ds(i * LANES, LANES)]
            idxg[0, pl.ds(i * LANES, LANES)] = (
                lax.shift_right_logical(dv, 4) + NACC)
            colb = (dv & 15) * 8
            er = lax.iota(jnp.int32, LANES) + i * LANES
            for c3 in range(3):
                val = plsc.load_gather(mxbuf, [er, jnp.full((LANES,), c3,
                                                            jnp.int32)])
                plsc.store_scatter(xbuf, [er, colb + c3], val)
            cols.append((er, colb))
        pltpu.sync_copy(xbuf, acc.at[idxg.at[0]], add=True)
        for er, colb in cols:
            for c3 in range(3):
                plsc.store_scatter(xbuf, [er, colb + c3], zero16)
        return carry

    lax.fori_loop(0, S_NCHUNK, chunk, 0)
    plsc.subcore_barrier()
    pltpu.sync_copy(acc.at[rows], out_hbm.at[c].at[rows])


def kernel(h, x, edge_index, mask_ligand, edge_attr, We1, be1, We2, be2,
           Winf, binf, Wx1, bx1, Wx2, Wn1, bn1, Wn2, bn2):
    xpad = jnp.pad(x, ((0, 0), (0, XW - 3)))
    dst3d = edge_index[1].reshape(NCHUNKS_ALL, NSUB, SUB)
    src3d = edge_index[0].reshape(NCHUNKS_ALL, NSUB, SUB)

    We1a = We1[:HID]
    We1b = We1[HID:2 * HID]
    We1d = We1[2 * HID:2 * HID + NUM_G]
    We1e = We1[2 * HID + NUM_G:]

    a, b = _tc_pre(h, We1a, We1b)
    ad, bs = _sc_gather(a, b, dst3d, src3d)
    rel = _sc_rel(x[:, 0], x[:, 1], x[:, 2], dst3d, src3d).reshape(E, XW)
    msg, xmsg = _tc_edge(ad, bs, rel, edge_attr, We1d, We1e,
                         be1.reshape(1, HID), We2, be2.reshape(1, HID),
                         Winf.T, binf.reshape(1, 1), Wx1,
                         bx1.reshape(1, HID), Wx2.T)
    zeros = jnp.zeros((NACC2, HID), jnp.float32)
    dst3d_s = edge_index[1].reshape(S_NCHUNKS_ALL, 1, S_CHUNK)
    parts = _sc_scatter(msg, xmsg, dst3d_s, zeros)
    parts_msg = parts[:, :N]
    parts_dx = parts[:, NACC:].reshape(NC, NACC, XW)[:, :N]
    mask_f = mask_ligand.astype(jnp.float32).reshape(N, 1)
    h_out, xout_pad = _tc_node(h, xpad, parts_msg, parts_dx, mask_f,
                               Wn1[:HID], Wn1[HID:], bn1.reshape(1, HID),
                               Wn2, bn2.reshape(1, HID))
    return h_out, xout_pad[:, :3]


# speedup vs baseline: 1.1115x; 1.1115x over previous
import functools

import jax
import jax.numpy as jnp
import numpy as np
from jax import lax
from jax.experimental import pallas as pl
from jax.experimental.pallas import tpu as pltpu
from jax.experimental.pallas import tpu_sc as plsc

N = 10000
E = 320000
HID = 128
XW = 8
NUM_G = 16
LANES = 16

KS = 2
ES = E // KS

NC = 2
NS = 16
NW = NC * NS
EPW = ES // NW

SUB = 100
NSUB = 2
CHUNK = SUB * NSUB
NCHUNK = EPW // CHUNK
NCHUNKS_ALL = ES // CHUNK

R_SUB = 80
R_NSUB = 5
R_CHUNK = R_SUB * R_NSUB
R_EPW = E // NW
R_NCHUNK = R_EPW // R_CHUNK
R_NCHUNKS_ALL = E // R_CHUNK

S_SUB = 40
S_NSUB = 5
S_CHUNK = S_SUB * S_NSUB
S_NCHUNK = EPW // S_CHUNK
S_NCHUNKS_ALL = ES // S_CHUNK

NACC = 10240
ROWS_PT = NACC // NS


def _pre_body(h_ref, wa_ref, wb_ref, a_ref, b_ref):
    h = h_ref[...]
    a_ref[...] = jnp.dot(h, wa_ref[...], preferred_element_type=jnp.float32)
    b_ref[...] = jnp.dot(h, wb_ref[...], preferred_element_type=jnp.float32)


def _tc_pre(h, We1a, We1b):
    return pl.pallas_call(
        _pre_body,
        out_shape=(jax.ShapeDtypeStruct((N, HID), jnp.float32),
                   jax.ShapeDtypeStruct((N, HID), jnp.float32)),
    )(h, We1a, We1b)


_EB = 2000
_EGRID = ES // _EB
_G_STEP = float(np.float32(10.0) / np.float32(NUM_G - 1))
_G_COEFF = float(-0.5 / np.linspace(0.0, 10.0, NUM_G)[1] ** 2)


def _edge_body(ad_ref, bs_ref, rel_ref, ea_ref, wd_ref, wea_ref, be1_ref,
               we2_ref, be2_ref, winf_ref, binf_ref, wx1_ref, bx1_ref,
               wx2_ref, msg_ref, xmsg_ref):
    t1pre = ad_ref[...] + bs_ref[...]
    rel = rel_ref[...]
    d_sq = jnp.sum(rel * rel, axis=1, keepdims=True)
    dist = jnp.sqrt(d_sq + 1e-8)
    offs = (lax.broadcasted_iota(jnp.int32, (1, NUM_G), 1)
            .astype(jnp.float32) * _G_STEP)
    dfeat = jnp.exp(_G_COEFF * (dist - offs) ** 2)
    t1 = (t1pre
          + jnp.dot(dfeat, wd_ref[...], preferred_element_type=jnp.float32)
          + be1_ref[...])
    ea = ea_ref[...]
    wea = wea_ref[...]
    for k in range(4):
        t1 = t1 + ea[:, k:k + 1] * wea[k:k + 1, :]
    u = t1 * jax.nn.sigmoid(t1)
    m1 = jnp.dot(u, we2_ref[...], preferred_element_type=jnp.float32) + be2_ref[...]
    mij = m1 * jax.nn.sigmoid(m1)
    eij = jax.nn.sigmoid(
        jnp.sum(mij * winf_ref[...], axis=1, keepdims=True) + binf_ref[...])
    v1 = jnp.dot(mij, wx1_ref[...], preferred_element_type=jnp.float32) + bx1_ref[...]
    v = v1 * jax.nn.sigmoid(v1)
    xg = jnp.tanh(jnp.sum(v * wx2_ref[...], axis=1, keepdims=True))
    xmsg = rel * (xg / (dist + 1.0))
    msg_ref[...] = mij * eij
    xmsg_ref[...] = jnp.concatenate(
        [xmsg, jnp.zeros((xmsg.shape[0], HID - XW), jnp.float32)], axis=1)


def _tc_edge(ad, bs, rel, edge_attr, We1d, We1e, be1, We2, be2, winf_row,
             binf, Wx1, bx1, wx2_row):
    full = lambda shape: pl.BlockSpec(shape, lambda i: (0, 0))
    return pl.pallas_call(
        _edge_body,
        grid=(_EGRID,),
        in_specs=[
            pl.BlockSpec((_EB, HID), lambda i: (i, 0)),
            pl.BlockSpec((_EB, HID), lambda i: (i, 0)),
            pl.BlockSpec((_EB, XW), lambda i: (i, 0)),
            pl.BlockSpec((_EB, 4), lambda i: (i, 0)),
            full((NUM_G, HID)),
            full((4, HID)),
            full((1, HID)),
            full((HID, HID)),
            full((1, HID)),
            full((1, HID)),
            full((1, 1)),
            full((HID, HID)),
            full((1, HID)),
            full((1, HID)),
        ],
        out_specs=(pl.BlockSpec((_EB, HID), lambda i: (i, 0)),
                   pl.BlockSpec((_EB, HID), lambda i: (i, 0))),
        out_shape=(jax.ShapeDtypeStruct((ES, HID), jnp.float32),
                   jax.ShapeDtypeStruct((ES, HID), jnp.float32)),
    )(ad, bs, rel, edge_attr, We1d, We1e, be1, We2, be2, winf_row, binf,
      Wx1, bx1, wx2_row)


def _node_body(h_ref, xp_ref, pm_ref, pd_ref, mask_ref, wn1a_ref, wn1b_ref,
               bn1_ref, wn2_ref, bn2_ref, hout_ref, xout_ref):
    h = h_ref[...]
    mi = pm_ref[0][:N] + pm_ref[1][:N]
    dx = pd_ref[0][:N, :XW] + pd_ref[1][:N, :XW]
    t1 = (jnp.dot(mi, wn1a_ref[...], preferred_element_type=jnp.float32)
          + jnp.dot(h, wn1b_ref[...], preferred_element_type=jnp.float32)
          + bn1_ref[...])
    t = t1 * jax.nn.sigmoid(t1)
    hout_ref[...] = h + jnp.dot(t, wn2_ref[...],
                                preferred_element_type=jnp.float32) + bn2_ref[...]
    xout_ref[...] = xp_ref[...] + dx * mask_ref[...]


def _tc_node(h, xpad, parts_msg, parts_dx, mask_f, Wn1a, Wn1b, bn1, Wn2, bn2):
    return pl.pallas_call(
        _node_body,
        out_shape=(jax.ShapeDtypeStruct((N, HID), jnp.float32),
                   jax.ShapeDtypeStruct((N, XW), jnp.float32)),
    )(h, xpad, parts_msg, parts_dx, mask_f, Wn1a, Wn1b, bn1, Wn2, bn2)


@functools.cache
def _sc_gather_kernel():
    mesh = plsc.VectorSubcoreMesh(core_axis_name="c", subcore_axis_name="s")
    return functools.partial(
        pl.kernel,
        mesh=mesh,
        out_type=(jax.ShapeDtypeStruct((ES, HID), jnp.float32),
                  jax.ShapeDtypeStruct((ES, HID), jnp.float32)),
        scratch_types=[
            pltpu.VMEM((NSUB, SUB), jnp.int32),
            pltpu.VMEM((NSUB, SUB), jnp.int32),
            pltpu.VMEM((CHUNK, HID), jnp.float32),
            pltpu.VMEM((CHUNK, HID), jnp.float32),
            pltpu.SemaphoreType.DMA,
        ],
    )(_sc_gather_body)


def _sc_gather(a, b, dst3d, src3d):
    return _sc_gather_kernel()(a, b, dst3d, src3d)


def _sc_gather_body(a_hbm, b_hbm, dst3d_hbm, src3d_hbm, ad_out, bs_out,
                    idxd, idxs, adb, bsb, sem):
    c = lax.axis_index("c")
    s = lax.axis_index("s")
    wid = s * NC + c
    g0 = wid * NCHUNK
    e0w = wid * EPW

    def chunk(k, carry):
        pltpu.sync_copy(dst3d_hbm.at[g0 + k], idxd)
        pltpu.sync_copy(src3d_hbm.at[g0 + k], idxs)
        copies = []
        for j in range(NSUB):
            copies.append(pltpu.async_copy(
                a_hbm.at[idxd.at[j]], adb.at[pl.ds(j * SUB, SUB)], sem))
            copies.append(pltpu.async_copy(
                b_hbm.at[idxs.at[j]], bsb.at[pl.ds(j * SUB, SUB)], sem))
        for cp in copies:
            cp.wait()
        e0 = e0w + k * CHUNK
        pltpu.sync_copy(adb, ad_out.at[pl.ds(e0, CHUNK)])
        pltpu.sync_copy(bsb, bs_out.at[pl.ds(e0, CHUNK)])
        return carry

    lax.fori_loop(0, NCHUNK, chunk, 0)


@functools.cache
def _sc_rel_kernel():
    mesh = plsc.VectorSubcoreMesh(core_axis_name="c", subcore_axis_name="s")
    return functools.partial(
        pl.kernel,
        mesh=mesh,
        compiler_params=pltpu.CompilerParams(needs_layout_passes=False),
        out_type=jax.ShapeDtypeStruct((E * XW,), jnp.float32),
        scratch_types=[
            pltpu.VMEM((N,), jnp.float32),
            pltpu.VMEM((N,), jnp.float32),
            pltpu.VMEM((N,), jnp.float32),
            pltpu.VMEM((R_NSUB, R_SUB), jnp.int32),
            pltpu.VMEM((R_NSUB, R_SUB), jnp.int32),
            pltpu.VMEM((R_CHUNK * XW,), jnp.float32),
            pltpu.SemaphoreType.DMA,
        ],
    )(_sc_rel_body)


def _sc_rel(x0, x1, x2, dst3d, src3d):
    return _sc_rel_kernel()(x0, x1, x2, dst3d, src3d)


def _sc_rel_body(x0_hbm, x1_hbm, x2_hbm, dst3d_hbm, src3d_hbm, rel_out,
                 x0b, x1b, x2b, idxd, idxs, relb, sem):
    c = lax.axis_index("c")
    s = lax.axis_index("s")
    wid = s * NC + c
    g0 = wid * R_NCHUNK
    e0w = wid * R_EPW
    pltpu.sync_copy(x0_hbm, x0b)
    pltpu.sync_copy(x1_hbm, x1b)
    pltpu.sync_copy(x2_hbm, x2b)

    def zero(v, carry):
        relb[pl.ds(v * LANES, LANES)] = jnp.zeros((LANES,), jnp.float32)
        return carry

    lax.fori_loop(0, R_CHUNK * XW // LANES, zero, 0)

    def chunk(k, carry):
        pltpu.sync_copy(dst3d_hbm.at[g0 + k], idxd)
        pltpu.sync_copy(src3d_hbm.at[g0 + k], idxs)
        for j in range(R_NSUB):
            for i in range(R_SUB // LANES):
                ivd = idxd[j, pl.ds(i * LANES, LANES)]
                ivs = idxs[j, pl.ds(i * LANES, LANES)]
                base = (j * R_SUB + i * LANES) * XW
                flat = lax.iota(jnp.int32, LANES) * XW + base
                for comp, xb in ((0, x0b), (1, x1b), (2, x2b)):
                    d = plsc.load_gather(xb, [ivd])
                    sv = plsc.load_gather(xb, [ivs])
                    plsc.store_scatter(relb, [flat + comp], d - sv)
        pltpu.sync_copy(relb, rel_out.at[pl.ds((e0w + k * R_CHUNK) * XW,
                                               R_CHUNK * XW)])
        return carry

    lax.fori_loop(0, R_NCHUNK, chunk, 0)


@functools.cache
def _sc_scatter_kernel():
    mesh = plsc.VectorSubcoreMesh(core_axis_name="c", subcore_axis_name="s")
    return functools.partial(
        pl.kernel,
        mesh=mesh,
        out_type=(jax.ShapeDtypeStruct((NC, NACC, HID), jnp.float32),
                  jax.ShapeDtypeStruct((NC, NACC, HID), jnp.float32)),
        scratch_types=[
            pltpu.VMEM((S_NSUB, S_SUB), jnp.int32),
            pltpu.VMEM((S_CHUNK, HID), jnp.float32),
            pltpu.VMEM_SHARED((NACC, HID), jnp.float32),
            pltpu.SemaphoreType.DMA,
        ],
    )(_sc_scatter_body)


def _sc_scatter(msg, xmsg, dst3d, pm_init, pd_init):
    return _sc_scatter_kernel()(msg, xmsg, dst3d, pm_init, pd_init)


def _sc_scatter_body(msg_hbm, xmsg_hbm, dst3d_hbm, pmi_hbm, pdi_hbm,
                     pm_hbm, pd_hbm, idxb, mbuf, acc, sem):
    c = lax.axis_index("c")
    s = lax.axis_index("s")
    wid = c * NS + s
    g0 = wid * S_NCHUNK
    e0w = wid * EPW
    rows = pl.ds(s * ROWS_PT, ROWS_PT)

    for src_hbm, init_hbm, out_hbm in ((msg_hbm, pmi_hbm, pm_hbm),
                                       (xmsg_hbm, pdi_hbm, pd_hbm)):
        pltpu.sync_copy(init_hbm.at[c].at[rows], acc.at[rows])
        plsc.subcore_barrier()

        def chunk(k, carry):
            pltpu.sync_copy(dst3d_hbm.at[g0 + k], idxb)
            pltpu.sync_copy(src_hbm.at[pl.ds(e0w + k * S_CHUNK, S_CHUNK)], mbuf)
            for j in range(S_NSUB):
                pltpu.sync_copy(mbuf.at[pl.ds(j * S_SUB, S_SUB)],
                                acc.at[idxb.at[j]], add=True)
            return carry

        lax.fori_loop(0, S_NCHUNK, chunk, 0)
        plsc.subcore_barrier()
        pltpu.sync_copy(acc.at[rows], out_hbm.at[c].at[rows])
        plsc.subcore_barrier()


def kernel(h, x, edge_index, mask_ligand, edge_attr, We1, be1, We2, be2,
           Winf, binf, Wx1, bx1, Wx2, Wn1, bn1, Wn2, bn2):
    xpad = jnp.pad(x, ((0, 0), (0, XW - 3)))
    src = edge_index[0]
    dst = edge_index[1]
    dst3d_r = dst.reshape(R_NCHUNKS_ALL, R_NSUB, R_SUB)
    src3d_r = src.reshape(R_NCHUNKS_ALL, R_NSUB, R_SUB)

    We1a = We1[:HID]
    We1b = We1[HID:2 * HID]
    We1d = We1[2 * HID:2 * HID + NUM_G]
    We1e = We1[2 * HID + NUM_G:]

    rel_flat = _sc_rel(x[:, 0], x[:, 1], x[:, 2], dst3d_r, src3d_r)
    rel = rel_flat.reshape(E, XW)
    a, b = _tc_pre(h, We1a, We1b)

    gath = []
    for k in range(KS):
        sl = slice(k * ES, (k + 1) * ES)
        dst3d = dst[sl].reshape(NCHUNKS_ALL, NSUB, SUB)
        src3d = src[sl].reshape(NCHUNKS_ALL, NSUB, SUB)
        gath.append(_sc_gather(a, b, dst3d, src3d))

    pm = jnp.zeros((NC, NACC, HID), jnp.float32)
    pd = jnp.zeros((NC, NACC, HID), jnp.float32)
    for k in range(KS):
        sl = slice(k * ES, (k + 1) * ES)
        ad, bs = gath[k]
        msg, xmsg = _tc_edge(ad, bs, rel[sl], edge_attr[sl], We1d, We1e,
                             be1.reshape(1, HID), We2, be2.reshape(1, HID),
                             Winf.T, binf.reshape(1, 1), Wx1,
                             bx1.reshape(1, HID), Wx2.T)
        dst3d_s = dst[sl].reshape(S_NCHUNKS_ALL, S_NSUB, S_SUB)
        pm, pd = _sc_scatter(msg, xmsg, dst3d_s, pm, pd)

    mask_f = mask_ligand.astype(jnp.float32).reshape(N, 1)
    h_out, xout_pad = _tc_node(h, xpad, pm, pd, mask_f,
                               Wn1[:HID], Wn1[HID:], bn1.reshape(1, HID),
                               Wn2, bn2.reshape(1, HID))
    return h_out, xout_pad[:, :3]


# speedup vs baseline: 1.2486x; 1.1233x over previous
import functools

import jax
import jax.numpy as jnp
import numpy as np
from jax import lax
from jax.experimental import pallas as pl
from jax.experimental.pallas import tpu as pltpu
from jax.experimental.pallas import tpu_sc as plsc

N = 10000
E = 320000
HID = 128
XW = 8
NUM_G = 16
LANES = 16

KS = 2
ES = E // KS

NC = 2
NS = 16
NW = NC * NS
EPW = ES // NW

SUB = 100
NSUB = 2
CHUNK = SUB * NSUB
NCHUNK = EPW // CHUNK
NCHUNKS_ALL = ES // CHUNK

R_SUB = 80
R_NSUB = 5
R_CHUNK = R_SUB * R_NSUB
R_EPW = E // NW
R_NCHUNK = R_EPW // R_CHUNK
R_NCHUNKS_ALL = E // R_CHUNK

S_SUB = 40
S_NSUB = 5
S_CHUNK = S_SUB * S_NSUB
S_NCHUNK = EPW // S_CHUNK
S_NCHUNKS_ALL = ES // S_CHUNK

NACC = 10240
ROWS_PT = NACC // NS


def _pre_body(h_ref, wa_ref, wb_ref, a_ref, b_ref):
    h = h_ref[...]
    a_ref[...] = jnp.dot(h, wa_ref[...], preferred_element_type=jnp.float32)
    b_ref[...] = jnp.dot(h, wb_ref[...], preferred_element_type=jnp.float32)


def _tc_pre(h, We1a, We1b):
    return pl.pallas_call(
        _pre_body,
        out_shape=(jax.ShapeDtypeStruct((N, HID), jnp.float32),
                   jax.ShapeDtypeStruct((N, HID), jnp.float32)),
    )(h, We1a, We1b)


_EB = 2000
_EGRID = ES // _EB
_G_STEP = float(np.float32(10.0) / np.float32(NUM_G - 1))
_G_COEFF = float(-0.5 / np.linspace(0.0, 10.0, NUM_G)[1] ** 2)


def _edge_body(ad_ref, bs_ref, rel_ref, ea_ref, wd_ref, wea_ref, be1_ref,
               we2_ref, be2_ref, winf_ref, binf_ref, wx1_ref, bx1_ref,
               wx2_ref, msg_ref, xmsg_ref):
    t1pre = ad_ref[...] + bs_ref[...]
    rel = rel_ref[...]
    d_sq = jnp.sum(rel * rel, axis=1, keepdims=True)
    dist = jnp.sqrt(d_sq + 1e-8)
    offs = (lax.broadcasted_iota(jnp.int32, (1, NUM_G), 1)
            .astype(jnp.float32) * _G_STEP)
    dfeat = jnp.exp(_G_COEFF * (dist - offs) ** 2)
    t1 = (t1pre
          + jnp.dot(dfeat, wd_ref[...], preferred_element_type=jnp.float32)
          + be1_ref[...])
    ea = ea_ref[...]
    wea = wea_ref[...]
    for k in range(4):
        t1 = t1 + ea[:, k:k + 1] * wea[k:k + 1, :]
    u = t1 * jax.nn.sigmoid(t1)
    m1 = jnp.dot(u, we2_ref[...], preferred_element_type=jnp.float32) + be2_ref[...]
    mij = m1 * jax.nn.sigmoid(m1)
    eij = jax.nn.sigmoid(
        jnp.sum(mij * winf_ref[...], axis=1, keepdims=True) + binf_ref[...])
    v1 = jnp.dot(mij, wx1_ref[...], preferred_element_type=jnp.float32) + bx1_ref[...]
    v = v1 * jax.nn.sigmoid(v1)
    xg = jnp.tanh(jnp.sum(v * wx2_ref[...], axis=1, keepdims=True))
    xmsg = rel * (xg / (dist + 1.0))
    msg_ref[...] = mij * eij
    xmsg_ref[...] = jnp.concatenate(
        [xmsg, jnp.zeros((xmsg.shape[0], HID - XW), jnp.float32)], axis=1)


def _tc_edge(k_slice, ad, bs, rel, edge_attr, We1d, We1e, be1, We2, be2,
             winf_row, binf, Wx1, bx1, wx2_row):
    full = lambda shape: pl.BlockSpec(shape, lambda i: (0, 0))
    off = k_slice * _EGRID
    return pl.pallas_call(
        _edge_body,
        grid=(_EGRID,),
        in_specs=[
            pl.BlockSpec((_EB, HID), lambda i: (i, 0)),
            pl.BlockSpec((_EB, HID), lambda i: (i, 0)),
            pl.BlockSpec((_EB, XW), lambda i: (i + off, 0)),
            pl.BlockSpec((_EB, 4), lambda i: (i + off, 0)),
            full((NUM_G, HID)),
            full((4, HID)),
            full((1, HID)),
            full((HID, HID)),
            full((1, HID)),
            full((1, HID)),
            full((1, 1)),
            full((HID, HID)),
            full((1, HID)),
            full((1, HID)),
        ],
        out_specs=(pl.BlockSpec((_EB, HID), lambda i: (i, 0)),
                   pl.BlockSpec((_EB, HID), lambda i: (i, 0))),
        out_shape=(jax.ShapeDtypeStruct((ES, HID), jnp.float32),
                   jax.ShapeDtypeStruct((ES, HID), jnp.float32)),
    )(ad, bs, rel, edge_attr, We1d, We1e, be1, We2, be2, winf_row, binf,
      Wx1, bx1, wx2_row)


def _node_body(h_ref, xp_ref, pm_ref, pd_ref, mask_ref, wn1a_ref, wn1b_ref,
               bn1_ref, wn2_ref, bn2_ref, hout_ref, xout_ref):
    h = h_ref[...]
    mi = pm_ref[0][:N] + pm_ref[1][:N]
    dx = pd_ref[0][:N, :XW] + pd_ref[1][:N, :XW]
    t1 = (jnp.dot(mi, wn1a_ref[...], preferred_element_type=jnp.float32)
          + jnp.dot(h, wn1b_ref[...], preferred_element_type=jnp.float32)
          + bn1_ref[...])
    t = t1 * jax.nn.sigmoid(t1)
    hout_ref[...] = h + jnp.dot(t, wn2_ref[...],
                                preferred_element_type=jnp.float32) + bn2_ref[...]
    xout_ref[...] = xp_ref[...] + dx * mask_ref[...]


def _tc_node(h, xpad, parts_msg, parts_dx, mask_f, Wn1a, Wn1b, bn1, Wn2, bn2):
    return pl.pallas_call(
        _node_body,
        out_shape=(jax.ShapeDtypeStruct((N, HID), jnp.float32),
                   jax.ShapeDtypeStruct((N, XW), jnp.float32)),
    )(h, xpad, parts_msg, parts_dx, mask_f, Wn1a, Wn1b, bn1, Wn2, bn2)


@functools.cache
def _sc_gather_kernel():
    mesh = plsc.VectorSubcoreMesh(core_axis_name="c", subcore_axis_name="s")
    return functools.partial(
        pl.kernel,
        mesh=mesh,
        out_type=(jax.ShapeDtypeStruct((ES, HID), jnp.float32),
                  jax.ShapeDtypeStruct((ES, HID), jnp.float32)),
        scratch_types=[
            pltpu.VMEM((NSUB, SUB), jnp.int32),
            pltpu.VMEM((NSUB, SUB), jnp.int32),
            pltpu.VMEM((CHUNK, HID), jnp.float32),
            pltpu.VMEM((CHUNK, HID), jnp.float32),
            pltpu.SemaphoreType.DMA,
        ],
    )(_sc_gather_body)


def _sc_gather(a, b, dst3d, src3d):
    return _sc_gather_kernel()(a, b, dst3d, src3d)


def _sc_gather_body(a_hbm, b_hbm, dst3d_hbm, src3d_hbm, ad_out, bs_out,
                    idxd, idxs, adb, bsb, sem):
    c = lax.axis_index("c")
    s = lax.axis_index("s")
    wid = s * NC + c
    g0 = wid * NCHUNK
    e0w = wid * EPW

    def chunk(k, carry):
        pltpu.sync_copy(dst3d_hbm.at[g0 + k], idxd)
        pltpu.sync_copy(src3d_hbm.at[g0 + k], idxs)
        copies = []
        for j in range(NSUB):
            copies.append(pltpu.async_copy(
                a_hbm.at[idxd.at[j]], adb.at[pl.ds(j * SUB, SUB)], sem))
            copies.append(pltpu.async_copy(
                b_hbm.at[idxs.at[j]], bsb.at[pl.ds(j * SUB, SUB)], sem))
        for cp in copies:
            cp.wait()
        e0 = e0w + k * CHUNK
        pltpu.sync_copy(adb, ad_out.at[pl.ds(e0, CHUNK)])
        pltpu.sync_copy(bsb, bs_out.at[pl.ds(e0, CHUNK)])
        return carry

    lax.fori_loop(0, NCHUNK, chunk, 0)


@functools.cache
def _sc_rel_kernel():
    mesh = plsc.VectorSubcoreMesh(core_axis_name="c", subcore_axis_name="s")
    return functools.partial(
        pl.kernel,
        mesh=mesh,
        compiler_params=pltpu.CompilerParams(needs_layout_passes=False),
        out_type=jax.ShapeDtypeStruct((E, XW), jnp.float32),
        scratch_types=[
            pltpu.VMEM((N,), jnp.float32),
            pltpu.VMEM((N,), jnp.float32),
            pltpu.VMEM((N,), jnp.float32),
            pltpu.VMEM((R_NSUB, R_SUB), jnp.int32),
            pltpu.VMEM((R_NSUB, R_SUB), jnp.int32),
            pltpu.VMEM((R_CHUNK, XW), jnp.float32),
            pltpu.SemaphoreType.DMA,
        ],
    )(_sc_rel_body)


def _sc_rel(x0, x1, x2, dst3d, src3d):
    return _sc_rel_kernel()(x0, x1, x2, dst3d, src3d)


def _sc_rel_body(x0_hbm, x1_hbm, x2_hbm, dst3d_hbm, src3d_hbm, rel_out,
                 x0b, x1b, x2b, idxd, idxs, relb, sem):
    c = lax.axis_index("c")
    s = lax.axis_index("s")
    wid = s * NC + c
    g0 = wid * R_NCHUNK
    e0w = wid * R_EPW
    pltpu.sync_copy(x0_hbm, x0b)
    pltpu.sync_copy(x1_hbm, x1b)
    pltpu.sync_copy(x2_hbm, x2b)

    lanes = lax.iota(jnp.int32, LANES)

    def zero(v, carry):
        flat = v * LANES + lanes
        plsc.store_scatter(relb, [lax.shift_right_logical(flat, 3),
                                  flat & 7], jnp.zeros((LANES,), jnp.float32))
        return carry

    lax.fori_loop(0, R_CHUNK * XW // LANES, zero, 0)

    def chunk(k, carry):
        pltpu.sync_copy(dst3d_hbm.at[g0 + k], idxd)
        pltpu.sync_copy(src3d_hbm.at[g0 + k], idxs)
        for j in range(R_NSUB):
            for i in range(R_SUB // LANES):
                ivd = idxd[j, pl.ds(i * LANES, LANES)]
                ivs = idxs[j, pl.ds(i * LANES, LANES)]
                rows = lanes + (j * R_SUB + i * LANES)
                for comp, xb in ((0, x0b), (1, x1b), (2, x2b)):
                    d = plsc.load_gather(xb, [ivd])
                    sv = plsc.load_gather(xb, [ivs])
                    plsc.store_scatter(relb, [rows, jnp.full((LANES,), comp,
                                                             jnp.int32)],
                                       d - sv)
        pltpu.sync_copy(relb, rel_out.at[pl.ds(e0w + k * R_CHUNK, R_CHUNK)])
        return carry

    lax.fori_loop(0, R_NCHUNK, chunk, 0)


@functools.cache
def _sc_scatter_kernel():
    mesh = plsc.VectorSubcoreMesh(core_axis_name="c", subcore_axis_name="s")
    return functools.partial(
        pl.kernel,
        mesh=mesh,
        out_type=(jax.ShapeDtypeStruct((NC, NACC, HID), jnp.float32),
                  jax.ShapeDtypeStruct((NC, NACC, HID), jnp.float32)),
        scratch_types=[
            pltpu.VMEM((S_NSUB, S_SUB), jnp.int32),
            pltpu.VMEM((S_CHUNK, HID), jnp.float32),
            pltpu.VMEM_SHARED((NACC, HID), jnp.float32),
            pltpu.SemaphoreType.DMA,
        ],
    )(_sc_scatter_body)


def _sc_scatter(msg, xmsg, dst3d, pm_init, pd_init):
    return _sc_scatter_kernel()(msg, xmsg, dst3d, pm_init, pd_init)


def _sc_scatter_body(msg_hbm, xmsg_hbm, dst3d_hbm, pmi_hbm, pdi_hbm,
                     pm_hbm, pd_hbm, idxb, mbuf, acc, sem):
    c = lax.axis_index("c")
    s = lax.axis_index("s")
    wid = c * NS + s
    g0 = wid * S_NCHUNK
    e0w = wid * EPW
    rows = pl.ds(s * ROWS_PT, ROWS_PT)

    for src_hbm, init_hbm, out_hbm in ((msg_hbm, pmi_hbm, pm_hbm),
                                       (xmsg_hbm, pdi_hbm, pd_hbm)):
        pltpu.sync_copy(init_hbm.at[c].at[rows], acc.at[rows])
        plsc.subcore_barrier()

        def chunk(k, carry):
            pltpu.sync_copy(dst3d_hbm.at[g0 + k], idxb)
            pltpu.sync_copy(src_hbm.at[pl.ds(e0w + k * S_CHUNK, S_CHUNK)], mbuf)
            for j in range(S_NSUB):
                pltpu.sync_copy(mbuf.at[pl.ds(j * S_SUB, S_SUB)],
                                acc.at[idxb.at[j]], add=True)
            return carry

        lax.fori_loop(0, S_NCHUNK, chunk, 0)
        plsc.subcore_barrier()
        pltpu.sync_copy(acc.at[rows], out_hbm.at[c].at[rows])
        plsc.subcore_barrier()


def kernel(h, x, edge_index, mask_ligand, edge_attr, We1, be1, We2, be2,
           Winf, binf, Wx1, bx1, Wx2, Wn1, bn1, Wn2, bn2):
    xpad = jnp.pad(x, ((0, 0), (0, XW - 3)))
    src = edge_index[0]
    dst = edge_index[1]
    dst3d_r = dst.reshape(R_NCHUNKS_ALL, R_NSUB, R_SUB)
    src3d_r = src.reshape(R_NCHUNKS_ALL, R_NSUB, R_SUB)

    We1a = We1[:HID]
    We1b = We1[HID:2 * HID]
    We1d = We1[2 * HID:2 * HID + NUM_G]
    We1e = We1[2 * HID + NUM_G:]

    rel = _sc_rel(x[:, 0], x[:, 1], x[:, 2], dst3d_r, src3d_r)
    a, b = _tc_pre(h, We1a, We1b)

    gath = []
    for k in range(KS):
        sl = slice(k * ES, (k + 1) * ES)
        dst3d = dst[sl].reshape(NCHUNKS_ALL, NSUB, SUB)
        src3d = src[sl].reshape(NCHUNKS_ALL, NSUB, SUB)
        gath.append(_sc_gather(a, b, dst3d, src3d))

    pm = jnp.zeros((NC, NACC, HID), jnp.float32)
    pd = jnp.zeros((NC, NACC, HID), jnp.float32)
    for k in range(KS):
        sl = slice(k * ES, (k + 1) * ES)
        ad, bs = gath[k]
        msg, xmsg = _tc_edge(k, ad, bs, rel, edge_attr, We1d, We1e,
                             be1.reshape(1, HID), We2, be2.reshape(1, HID),
                             Winf.T, binf.reshape(1, 1), Wx1,
                             bx1.reshape(1, HID), Wx2.T)
        dst3d_s = dst[sl].reshape(S_NCHUNKS_ALL, S_NSUB, S_SUB)
        pm, pd = _sc_scatter(msg, xmsg, dst3d_s, pm, pd)

    mask_f = mask_ligand.astype(jnp.float32).reshape(N, 1)
    h_out, xout_pad = _tc_node(h, xpad, pm, pd, mask_f,
                               Wn1[:HID], Wn1[HID:], bn1.reshape(1, HID),
                               Wn2, bn2.reshape(1, HID))
    return h_out, xout_pad[:, :3]
